# baseline (speedup 1.0000x reference)
import jax
import jax.numpy as jnp
from jax import lax
from jax.experimental import pallas as pl
from jax.experimental.pallas import tpu as pltpu
from jax.experimental.pallas import tpu_sc as plsc

_BATCH = 16384
_DIM = 64
_NC = 2
_NS = 16
_NW = _NC * _NS
_BPW = _BATCH // _NW


def _gather_body(table_hbm, idx_hbm, out_hbm, idx_v, out_v, sem):
    wid = lax.axis_index("s") * _NC + lax.axis_index("c")
    base = wid * _BPW

    pltpu.sync_copy(idx_hbm.at[pl.ds(base, _BPW)], idx_v)

    def fire(g, _):
        vec = idx_v[pl.ds(g * 16, 16)]
        for j in range(16):
            pltpu.async_copy(
                table_hbm.at[vec[j]],
                out_v.at[g * 16 + j],
                sem,
            )
        return 0

    lax.fori_loop(0, _BPW // 16, fire, 0)

    def drain(r, _):
        pltpu.make_async_copy(table_hbm.at[0], out_v.at[0], sem).wait()
        return 0

    lax.fori_loop(0, _BPW, drain, 0, unroll=8)

    pltpu.sync_copy(out_v, out_hbm.at[pl.ds(base, _BPW)])


@jax.jit
def kernel(idx, latent_codes):
    run = pl.kernel(
        _gather_body,
        mesh=plsc.VectorSubcoreMesh(core_axis_name="c", subcore_axis_name="s"),
        out_type=jax.ShapeDtypeStruct((_BATCH, _DIM), jnp.float32),
        scratch_types=[
            pltpu.VMEM((_BPW,), jnp.int32),
            pltpu.VMEM((_BPW, _DIM), jnp.float32),
            pltpu.SemaphoreType.DMA,
        ],
        compiler_params=pltpu.CompilerParams(use_tc_tiling_on_sc=True),
    )
    return run(latent_codes, idx.astype(jnp.int32))


# speedup vs baseline: 1.1663x; 1.1663x over previous
import jax
import jax.numpy as jnp
from jax import lax
from jax.experimental import pallas as pl
from jax.experimental.pallas import tpu as pltpu
from jax.experimental.pallas import tpu_sc as plsc

_BATCH = 16384
_DIM = 64
_NC = 2
_NS = 16
_NW = _NC * _NS
_BPW = _BATCH // _NW
_V = 1_000_000
_NBLK = 7813
_LASTB = _NBLK - 1
_TAIL0 = _LASTB * 128
_TMP = _BATCH * _DIM + _DIM
_RING = 32
_SLACK = 24


def _scan_body(tbl, idx_hbm, tail_hbm, tmp_hbm, idxa, lv, ov, buf0, buf1,
               ext, tailv, semA, semC):
    wid = lax.axis_index("s") * _NC + lax.axis_index("c")
    start = jnp.where(wid < 5, 245 * wid, 1225 + 244 * (wid - 5)).astype(jnp.int32)
    n = jnp.where(wid < 5, 245, 244).astype(jnp.int32)
    lo = start * 128
    hi = jnp.minimum((start + n) * 128, _V)

    pltpu.sync_copy(idx_hbm, idxa)
    @pl.when(wid == _NW - 1)
    def _():
        pltpu.sync_copy(tail_hbm, tailv)

    iota16 = lax.iota(jnp.int32, 16)

    def mscan(g, cnt):
        vec = idxa[pl.ds(g * 16, 16)]
        m = (vec >= lo) & (vec < hi)
        pc = plsc.all_reduce_population_count(m)[0]
        plsc.store_compressed(lv.at[pl.ds(cnt, 16)], vec, mask=m)
        plsc.store_compressed(ov.at[pl.ds(cnt, 16)], iota16 + g * 16, mask=m)
        return cnt + pc

    cnt = lax.fori_loop(0, _BATCH // 16, mscan, jnp.int32(0))
    lv[pl.ds(cnt, 16)] = jnp.full((16,), 2**30, jnp.int32)
    cnt16 = lax.div(cnt + 15, jnp.int32(16))

    def _n_stream(c):
        bstart = start + 4 * c
        nb = jnp.clip(n - 4 * c, 0, 4)
        haspart = jnp.logical_and(nb > 0, bstart + nb - 1 == _LASTB)
        return nb - haspart.astype(jnp.int32)

    def start_chunk(c, buf):
        bstart = start + 4 * c

        def one(k, _):
            b = bstart + k
            pltpu.async_copy(tbl.at[:, pl.ds(b * 128, 128)],
                             buf.at[:, pl.ds(k * 128, 128)], semA)
            return 0

        lax.fori_loop(0, _n_stream(c), one, 0)

    def wait_chunk(c):
        def w(i, _):
            pltpu.make_async_copy(tbl.at[:, pl.ds(0, 128)],
                                  buf0.at[:, pl.ds(0, 128)], semA).wait()
            return 0

        lax.fori_loop(0, _n_stream(c), w, 0)

    def process_chunk(c, buf, carry):
        clo = lo + 512 * c
        chi = jnp.minimum(clo + 512, hi)

        def fire_entry(args):
            vj, o, fired, drained = args

            def dr(d):
                pltpu.make_async_copy(tmp_hbm.at[pl.ds(0, _DIM)],
                                      ext.at[0], semC).wait()
                return d + 1

            drained = lax.cond(fired - drained >= _SLACK, dr,
                               lambda d: d, drained)
            slot = lax.rem(fired, jnp.int32(_RING))

            def from_buf(s):
                pvec = jnp.full((16,), jnp.int32(0)) + (vj - clo)
                for qq in range(4):
                    vals = plsc.load_gather(buf, [iota16 + 16 * qq, pvec])
                    ext[s, pl.ds(16 * qq, 16)] = vals
                return 0

            def from_tail(s):
                pvec = jnp.full((16,), jnp.int32(0)) + (vj - _TAIL0)
                for qq in range(4):
                    vals = plsc.load_gather(tailv, [iota16 + 16 * qq, pvec])
                    ext[s, pl.ds(16 * qq, 16)] = vals
                return 0

            lax.cond(vj >= _TAIL0, from_tail, from_buf, slot)
            pltpu.async_copy(ext.at[slot], tmp_hbm.at[pl.ds(o * _DIM, _DIM)],
                             semC)
            return fired + 1, drained

        def grp(q, carry):
            vv = lv[pl.ds(q * 16, 16)]
            m = (vv >= clo) & (vv < chi)
            pc = plsc.all_reduce_population_count(m)[0]

            def hit(carry):
                fired, drained = carry
                oo = ov[pl.ds(q * 16, 16)]
                for j in range(16):
                    vj = vv[j]
                    fired, drained = lax.cond(
                        (vj >= clo) & (vj < chi),
                        lambda fd, vj=vj, j=j: fire_entry(
                            (vj, oo[j], fd[0], fd[1])),
                        lambda fd: fd,
                        (fired, drained),
                    )
                return fired, drained

            return lax.cond(pc > 0, hit, lambda cr: cr, carry)

        return lax.fori_loop(0, cnt16, grp, carry)

    start_chunk(jnp.int32(0), buf0)
    start_chunk(jnp.int32(1), buf1)

    def pair(i, carry):
        c0 = 2 * i
        wait_chunk(c0)
        carry = process_chunk(c0, buf0, carry)
        start_chunk(c0 + 2, buf0)
        c1 = c0 + 1
        wait_chunk(c1)
        carry = process_chunk(c1, buf1, carry)
        start_chunk(c1 + 2, buf1)
        return carry

    fired, drained = lax.fori_loop(0, 31, pair,
                                   (jnp.int32(0), jnp.int32(0)))

    def fd(i, _):
        pltpu.make_async_copy(tmp_hbm.at[pl.ds(0, _DIM)], ext.at[0],
                              semC).wait()
        return 0

    lax.fori_loop(0, fired - drained, fd, 0)


def _perm_body(tmp_hbm, out_hbm, buf, otv):
    wid = lax.axis_index("s") * _NC + lax.axis_index("c")
    base = wid * _BPW
    pltpu.sync_copy(tmp_hbm.at[pl.ds(base * _DIM, _BPW * _DIM)], buf)
    iota16 = lax.iota(jnp.int32, 16)

    def tr(i, _):
        c = lax.div(i, jnp.int32(_BPW // 16))
        q = lax.rem(i, jnp.int32(_BPW // 16))
        idxv = (iota16 + q * 16) * _DIM + c
        otv[c, pl.ds(q * 16, 16)] = plsc.load_gather(buf, [idxv])
        return 0

    lax.fori_loop(0, _DIM * (_BPW // 16), tr, 0)
    pltpu.sync_copy(otv, out_hbm.at[:, pl.ds(base, _BPW)])


@jax.jit
def kernel(idx, latent_codes):
    mesh = plsc.VectorSubcoreMesh(core_axis_name="c", subcore_axis_name="s")
    params = pltpu.CompilerParams(use_tc_tiling_on_sc=True,
                                  needs_layout_passes=False)
    run_a = pl.kernel(
        _scan_body,
        mesh=mesh,
        out_type=jax.ShapeDtypeStruct((_TMP,), jnp.float32),
        scratch_types=[
            pltpu.VMEM((_BATCH,), jnp.int32),
            pltpu.VMEM((_BATCH + 16,), jnp.int32),
            pltpu.VMEM((_BATCH + 16,), jnp.int32),
            pltpu.VMEM((_DIM, 512), jnp.float32),
            pltpu.VMEM((_DIM, 512), jnp.float32),
            pltpu.VMEM((_RING, _DIM), jnp.float32),
            pltpu.VMEM((_DIM, _DIM), jnp.float32),
            pltpu.SemaphoreType.DMA,
            pltpu.SemaphoreType.DMA,
        ],
        compiler_params=params,
    )
    tail = latent_codes[_TAIL0:].T
    tmp = run_a(latent_codes.T, idx.astype(jnp.int32), tail)
    run_b = pl.kernel(
        _perm_body,
        mesh=mesh,
        out_type=jax.ShapeDtypeStruct((_DIM, _BATCH), jnp.float32),
        scratch_types=[
            pltpu.VMEM((_BPW * _DIM,), jnp.float32),
            pltpu.VMEM((_DIM, _BPW), jnp.float32),
        ],
        compiler_params=params,
    )
    return run_b(tmp).T


# speedup vs baseline: 3.2044x; 2.7475x over previous
import jax
import jax.numpy as jnp
from jax import lax
from jax.experimental import pallas as pl
from jax.experimental.pallas import tpu as pltpu
from jax.experimental.pallas import tpu_sc as plsc

_BATCH = 16384
_DIM = 64
_NC = 2
_NS = 16
_NW = _NC * _NS
_BPW = _BATCH // _NW
_V = 1_000_000
_NBLK = 7813
_LASTB = _NBLK - 1
_TAIL0 = _LASTB * 128
_TMP = _BATCH * _DIM + _DIM
_RING = 32
_SLACK = 24
_NCH = 62
_BCAP = 64


def _scan_body(tbl, idx_hbm, tail_hbm, tmp_hbm, idxa, bkt_v, bkt_o,
               stg_v, stg_o, buf0, buf1, ext, tailv, bcnt, semA, semC):
    wid = lax.axis_index("s") * _NC + lax.axis_index("c")
    start = jnp.where(wid < 5, 245 * wid, 1225 + 244 * (wid - 5)).astype(jnp.int32)
    n = jnp.where(wid < 5, 245, 244).astype(jnp.int32)
    lo = start * 128
    hi = jnp.minimum((start + n) * 128, _V)

    def zero(i, _):
        bcnt[i] = jnp.int32(0)
        return 0

    lax.fori_loop(0, _NCH + 1, zero, 0)

    pltpu.sync_copy(idx_hbm, idxa)
    @pl.when(wid == _NW - 1)
    def _():
        pltpu.sync_copy(tail_hbm, tailv)

    iota16 = lax.iota(jnp.int32, 16)
    lane0 = iota16 == 0

    def mscan(g, _):
        vec = idxa[pl.ds(g * 16, 16)]
        m = (vec >= lo) & (vec < hi)
        pc = plsc.all_reduce_population_count(m)[0]

        def dohits(_):
            plsc.store_compressed(stg_v.at[pl.ds(0, 16)], vec, mask=m)
            plsc.store_compressed(stg_o.at[pl.ds(0, 16)], iota16 + g * 16,
                                  mask=m)

            def app(e, _):
                v = stg_v[pl.ds(e, 16)][0]
                o = stg_o[pl.ds(e, 16)][0]
                ch = lax.shift_right_logical(v - lo, 9)
                c = bcnt[ch]

                @pl.when(c < _BCAP)
                def _():
                    pos = ch * _BCAP + c
                    plsc.store_compressed(
                        bkt_v.at[pl.ds(pos, 16)],
                        jnp.full((16,), 0, jnp.int32) + v, mask=lane0)
                    plsc.store_compressed(
                        bkt_o.at[pl.ds(pos, 16)],
                        jnp.full((16,), 0, jnp.int32) + o, mask=lane0)
                    bcnt[ch] = c + 1

                @pl.when(c >= _BCAP)
                def _():
                    bcnt[_NCH] = jnp.int32(1)

                return 0

            lax.fori_loop(0, pc, app, 0)
            return 0

        lax.cond(pc > 0, dohits, lambda _: 0, 0)
        return 0

    lax.fori_loop(0, _BATCH // 16, mscan, 0)
    ovf = bcnt[_NCH]

    def _n_stream(c):
        bstart = start + 4 * c
        nb = jnp.clip(n - 4 * c, 0, 4)
        haspart = jnp.logical_and(nb > 0, bstart + nb - 1 == _LASTB)
        return nb - haspart.astype(jnp.int32)

    def start_chunk(c, buf):
        bstart = start + 4 * c

        def one(k, _):
            b = bstart + k
            pltpu.async_copy(tbl.at[:, pl.ds(b * 128, 128)],
                             buf.at[:, pl.ds(k * 128, 128)], semA)
            return 0

        lax.fori_loop(0, _n_stream(c), one, 0)

    def wait_chunk(c):
        def w(i, _):
            pltpu.make_async_copy(tbl.at[:, pl.ds(0, 128)],
                                  buf0.at[:, pl.ds(0, 128)], semA).wait()
            return 0

        lax.fori_loop(0, _n_stream(c), w, 0)

    def process_chunk(c, buf, carry):
        clo = lo + 512 * c
        chi = jnp.minimum(clo + 512, hi)

        def fire_entry(vj, o, fired, drained):
            def dr(d):
                pltpu.make_async_copy(tmp_hbm.at[pl.ds(0, _DIM)],
                                      ext.at[0], semC).wait()
                return d + 1

            drained = lax.cond(fired - drained >= _SLACK, dr,
                               lambda d: d, drained)
            slot = lax.rem(fired, jnp.int32(_RING))

            def from_buf(s):
                pvec = jnp.full((16,), jnp.int32(0)) + (vj - clo)
                for qq in range(4):
                    vals = plsc.load_gather(buf, [iota16 + 16 * qq, pvec])
                    ext[s, pl.ds(16 * qq, 16)] = vals
                return 0

            def from_tail(s):
                pvec = jnp.full((16,), jnp.int32(0)) + (vj - _TAIL0)
                for qq in range(4):
                    vals = plsc.load_gather(tailv, [iota16 + 16 * qq, pvec])
                    ext[s, pl.ds(16 * qq, 16)] = vals
                return 0

            lax.cond(vj >= _TAIL0, from_tail, from_buf, slot)
            pltpu.async_copy(ext.at[slot], tmp_hbm.at[pl.ds(o * _DIM, _DIM)],
                             semC)
            return fired + 1, drained

        def fast(carry):
            nbk = bcnt[c]

            def fe(e, cr):
                v = bkt_v[pl.ds(c * _BCAP + e, 16)][0]
                o = bkt_o[pl.ds(c * _BCAP + e, 16)][0]
                return fire_entry(v, o, cr[0], cr[1])

            return lax.fori_loop(0, nbk, fe, carry)

        def slow(carry):
            def grp(q, cr):
                vec = idxa[pl.ds(q * 16, 16)]
                m = (vec >= clo) & (vec < chi)
                pc = plsc.all_reduce_population_count(m)[0]

                def hit(cr):
                    plsc.store_compressed(stg_v.at[pl.ds(0, 16)], vec,
                                          mask=m)
                    plsc.store_compressed(stg_o.at[pl.ds(0, 16)],
                                          iota16 + q * 16, mask=m)

                    def app(e, cr2):
                        v = stg_v[pl.ds(e, 16)][0]
                        o = stg_o[pl.ds(e, 16)][0]
                        return fire_entry(v, o, cr2[0], cr2[1])

                    return lax.fori_loop(0, pc, app, cr)

                return lax.cond(pc > 0, hit, lambda x: x, cr)

            return lax.fori_loop(0, _BATCH // 16, grp, carry)

        return lax.cond(ovf > 0, slow, fast, carry)

    start_chunk(jnp.int32(0), buf0)
    start_chunk(jnp.int32(1), buf1)

    def pair(i, carry):
        c0 = 2 * i
        wait_chunk(c0)
        carry = process_chunk(c0, buf0, carry)
        start_chunk(c0 + 2, buf0)
        c1 = c0 + 1
        wait_chunk(c1)
        carry = process_chunk(c1, buf1, carry)
        start_chunk(c1 + 2, buf1)
        return carry

    fired, drained = lax.fori_loop(0, 31, pair,
                                   (jnp.int32(0), jnp.int32(0)))

    def fd(i, _):
        pltpu.make_async_copy(tmp_hbm.at[pl.ds(0, _DIM)], ext.at[0],
                              semC).wait()
        return 0

    lax.fori_loop(0, fired - drained, fd, 0)


def _perm_body(tmp_hbm, out_hbm, buf, otv):
    wid = lax.axis_index("s") * _NC + lax.axis_index("c")
    base = wid * _BPW
    pltpu.sync_copy(tmp_hbm.at[pl.ds(base * _DIM, _BPW * _DIM)], buf)
    iota16 = lax.iota(jnp.int32, 16)

    def tr(c, _):
        for q in range(_BPW // 16):
            idxv = (iota16 + q * 16) * _DIM + c
            otv[c, pl.ds(q * 16, 16)] = plsc.load_gather(buf, [idxv])
        return 0

    lax.fori_loop(0, _DIM, tr, 0)
    pltpu.sync_copy(otv, out_hbm.at[:, pl.ds(base, _BPW)])


@jax.jit
def kernel(idx, latent_codes):
    mesh = plsc.VectorSubcoreMesh(core_axis_name="c", subcore_axis_name="s")
    params = pltpu.CompilerParams(use_tc_tiling_on_sc=True,
                                  needs_layout_passes=False)
    run_a = pl.kernel(
        _scan_body,
        mesh=mesh,
        out_type=jax.ShapeDtypeStruct((_TMP,), jnp.float32),
        scratch_types=[
            pltpu.VMEM((_BATCH,), jnp.int32),
            pltpu.VMEM((_NCH * _BCAP + 16,), jnp.int32),
            pltpu.VMEM((_NCH * _BCAP + 16,), jnp.int32),
            pltpu.VMEM((32,), jnp.int32),
            pltpu.VMEM((32,), jnp.int32),
            pltpu.VMEM((_DIM, 512), jnp.float32),
            pltpu.VMEM((_DIM, 512), jnp.float32),
            pltpu.VMEM((_RING, _DIM), jnp.float32),
            pltpu.VMEM((_DIM, _DIM), jnp.float32),
            pltpu.SMEM((_NCH + 1,), jnp.int32),
            pltpu.SemaphoreType.DMA,
            pltpu.SemaphoreType.DMA,
        ],
        compiler_params=params,
    )
    tail = latent_codes[_TAIL0:].T
    tmp = run_a(latent_codes.T, idx.astype(jnp.int32), tail)
    run_b = pl.kernel(
        _perm_body,
        mesh=mesh,
        out_type=jax.ShapeDtypeStruct((_DIM, _BATCH), jnp.float32),
        scratch_types=[
            pltpu.VMEM((_BPW * _DIM,), jnp.float32),
            pltpu.VMEM((_DIM, _BPW), jnp.float32),
        ],
        compiler_params=params,
    )
    return run_b(tmp).T
